# Initial kernel scaffold; baseline (speedup 1.0000x reference)
import jax
import jax.numpy as jnp
from jax.experimental import pallas as pl


def kernel(x, edge_index, params):
    raise NotImplementedError("write your pallas kernel here")


# speedup vs baseline: 1.1408x; 1.1408x over previous
import functools

import jax
import jax.numpy as jnp
from jax import lax
from jax.experimental import pallas as pl
from jax.experimental.pallas import tpu as pltpu

_N = 10000
_D = 128
_NP = 10240
_BR = 1024
_GRID = _NP // _BR
_EPS = 1e-5


def _mm_stats(X, w_ref, b_ref, y_ref, s_ref, ss_ref):
    Y = lax.dot_general(X, w_ref[...], (((1,), (0,)), ((), ())),
                        precision=lax.Precision.HIGHEST) + b_ref[...]
    y_ref[...] = Y
    i = pl.program_id(0)
    rid = lax.broadcasted_iota(jnp.int32, (_BR, 1), 0) + i * _BR
    Ym = jnp.where(rid < _N, Y, 0.0)

    @pl.when(i == 0)
    def _():
        s_ref[...] = jnp.zeros_like(s_ref)
        ss_ref[...] = jnp.zeros_like(ss_ref)

    s_ref[...] += jnp.sum(Ym, axis=0, keepdims=True)
    ss_ref[...] += jnp.sum(Ym * Ym, axis=0, keepdims=True)


def _k_sum3(p0, p1, x, w, b, y, s, ss):
    X = p0[...] + p1[...] - x[...]
    _mm_stats(X, w, b, y, s, ss)


def _k_aff(yin, a, c, w, b, y, s, ss):
    X = jnp.maximum(yin[...] * a[...] + c[...], 0.0)
    _mm_stats(X, w, b, y, s, ss)


def _k_dual(p0, p1, x, q0, q1, h2, wa, wb, b, y, s, ss):
    X1 = p0[...] + p1[...] - x[...]
    X2 = q0[...] + q1[...] - h2[...]
    Y = (lax.dot_general(X1, wa[...], (((1,), (0,)), ((), ())),
                         precision=lax.Precision.HIGHEST)
         + lax.dot_general(X2, wb[...], (((1,), (0,)), ((), ())),
                           precision=lax.Precision.HIGHEST) + b[...])
    y_ref, s_ref, ss_ref = y, s, ss
    y_ref[...] = Y
    i = pl.program_id(0)
    rid = lax.broadcasted_iota(jnp.int32, (_BR, 1), 0) + i * _BR
    Ym = jnp.where(rid < _N, Y, 0.0)

    @pl.when(i == 0)
    def _():
        s_ref[...] = jnp.zeros_like(s_ref)
        ss_ref[...] = jnp.zeros_like(ss_ref)

    s_ref[...] += jnp.sum(Ym, axis=0, keepdims=True)
    ss_ref[...] += jnp.sum(Ym * Ym, axis=0, keepdims=True)


def _k_out(yin, a, c, o):
    o[...] = jnp.maximum(yin[...] * a[...] + c[...], 0.0)


_ROWS = lambda: pl.BlockSpec((_BR, _D), lambda i: (i, 0))
_WMAT = lambda: pl.BlockSpec((_D, _D), lambda i: (0, 0))
_VEC = lambda: pl.BlockSpec((1, _D), lambda i: (0, 0))

_MM_OUT = lambda: (
    [jax.ShapeDtypeStruct((_NP, _D), jnp.float32),
     jax.ShapeDtypeStruct((1, _D), jnp.float32),
     jax.ShapeDtypeStruct((1, _D), jnp.float32)],
    [_ROWS(), _VEC(), _VEC()],
)


def _call_sum3(p0, p1, x, w, b):
    out_shape, out_specs = _MM_OUT()
    return pl.pallas_call(
        _k_sum3, grid=(_GRID,),
        in_specs=[_ROWS(), _ROWS(), _ROWS(), _WMAT(), _VEC()],
        out_specs=out_specs, out_shape=out_shape,
    )(p0, p1, x, w, b)


def _call_aff(yin, a, c, w, b):
    out_shape, out_specs = _MM_OUT()
    return pl.pallas_call(
        _k_aff, grid=(_GRID,),
        in_specs=[_ROWS(), _VEC(), _VEC(), _WMAT(), _VEC()],
        out_specs=out_specs, out_shape=out_shape,
    )(yin, a, c, w, b)


def _call_dual(p0, p1, x, q0, q1, h2, wa, wb, b):
    out_shape, out_specs = _MM_OUT()
    return pl.pallas_call(
        _k_dual, grid=(_GRID,),
        in_specs=[_ROWS()] * 6 + [_WMAT(), _WMAT(), _VEC()],
        out_specs=out_specs, out_shape=out_shape,
    )(p0, p1, x, q0, q1, h2, wa, wb, b)


def _call_out(yin, a, c):
    return pl.pallas_call(
        _k_out, grid=(_GRID,),
        in_specs=[_ROWS(), _VEC(), _VEC()],
        out_specs=_ROWS(),
        out_shape=jax.ShapeDtypeStruct((_NP, _D), jnp.float32),
    )(yin, a, c)


def _affine(s, ss, g, be):
    mean = s[0] / _N
    var = ss[0] / _N - mean * mean
    scale = g / jnp.sqrt(var + _EPS)
    shift = be - mean * scale
    return scale.reshape(1, _D), shift.reshape(1, _D)


def _agg_xla(h, src, dst):
    msg = jnp.take(h, src, axis=0)
    return h + jax.ops.segment_sum(msg, dst, num_segments=h.shape[0])


def kernel(x, edge_index, params):
    p = params
    src = edge_index[0]
    dst = edge_index[1]

    pad = ((0, _NP - _N), (0, 0))
    xp = jnp.pad(x, pad)

    a1 = _agg_xla(x, src, dst)
    p0 = jnp.pad(a1, pad)
    p1 = xp

    b = lambda k: p[k].reshape(1, _D)

    y1, s1, ss1 = _call_sum3(p0, p1, xp, p['W1'], b('b1'))
    sc1, sh1 = _affine(s1, ss1, p['g1'], p['be1'])
    y2, s2, ss2 = _call_aff(y1, sc1, sh1, p['W2'], b('b2'))
    sc2, sh2 = _affine(s2, ss2, p['g2'], p['be2'])
    h2 = _call_out(y2, sc2, sh2)

    a2 = _agg_xla(h2[:_N], src, dst)
    q0 = jnp.pad(a2, pad)
    q1 = h2

    y3, s3, ss3 = _call_dual(p0, p1, xp, q0, q1, h2,
                             p['W3'][:_D], p['W3'][_D:], b('b3'))
    sc3, sh3 = _affine(s3, ss3, p['g3'], p['be3'])
    y4, s4, ss4 = _call_aff(y3, sc3, sh3, p['W4'], b('b4'))
    sc4, sh4 = _affine(s4, ss4, p['g4'], p['be4'])
    y5, s5, ss5 = _call_aff(y4, sc4, sh4, p['W5'], b('b5'))
    sc5, sh5 = _affine(s5, ss5, p['g5'], p['be5'])
    out = _call_out(y5, sc5, sh5)
    return out[:_N]


# speedup vs baseline: 4.2315x; 3.7092x over previous
import functools

import jax
import jax.numpy as jnp
from jax import lax
from jax.experimental import pallas as pl
from jax.experimental.pallas import tpu as pltpu

_N = 10000
_D = 128
_NP = 10240
_BR = 1024
_GRID = _NP // _BR
_EPS = 1e-5


def _mm_stats(X, w_ref, b_ref, y_ref, s_ref, ss_ref):
    Y = lax.dot_general(X, w_ref[...], (((1,), (0,)), ((), ())),
                        precision=lax.Precision.HIGHEST) + b_ref[...]
    y_ref[...] = Y
    i = pl.program_id(0)
    rid = lax.broadcasted_iota(jnp.int32, (_BR, 1), 0) + i * _BR
    Ym = jnp.where(rid < _N, Y, 0.0)

    @pl.when(i == 0)
    def _():
        s_ref[...] = jnp.zeros_like(s_ref)
        ss_ref[...] = jnp.zeros_like(ss_ref)

    s_ref[...] += jnp.sum(Ym, axis=0, keepdims=True)
    ss_ref[...] += jnp.sum(Ym * Ym, axis=0, keepdims=True)


def _k_sum3(p0, p1, x, w, b, y, s, ss):
    X = p0[...] + p1[...] - x[...]
    _mm_stats(X, w, b, y, s, ss)


def _k_aff(yin, a, c, w, b, y, s, ss):
    X = jnp.maximum(yin[...] * a[...] + c[...], 0.0)
    _mm_stats(X, w, b, y, s, ss)


def _k_dual(p0, p1, x, q0, q1, h2, wa, wb, b, y, s, ss):
    X1 = p0[...] + p1[...] - x[...]
    X2 = q0[...] + q1[...] - h2[...]
    Y = (lax.dot_general(X1, wa[...], (((1,), (0,)), ((), ())),
                         precision=lax.Precision.HIGHEST)
         + lax.dot_general(X2, wb[...], (((1,), (0,)), ((), ())),
                           precision=lax.Precision.HIGHEST) + b[...])
    y_ref, s_ref, ss_ref = y, s, ss
    y_ref[...] = Y
    i = pl.program_id(0)
    rid = lax.broadcasted_iota(jnp.int32, (_BR, 1), 0) + i * _BR
    Ym = jnp.where(rid < _N, Y, 0.0)

    @pl.when(i == 0)
    def _():
        s_ref[...] = jnp.zeros_like(s_ref)
        ss_ref[...] = jnp.zeros_like(ss_ref)

    s_ref[...] += jnp.sum(Ym, axis=0, keepdims=True)
    ss_ref[...] += jnp.sum(Ym * Ym, axis=0, keepdims=True)


def _k_out(yin, a, c, o):
    o[...] = jnp.maximum(yin[...] * a[...] + c[...], 0.0)


_ROWS = lambda: pl.BlockSpec((_BR, _D), lambda i: (i, 0))
_WMAT = lambda: pl.BlockSpec((_D, _D), lambda i: (0, 0))
_VEC = lambda: pl.BlockSpec((1, _D), lambda i: (0, 0))

_MM_OUT = lambda: (
    [jax.ShapeDtypeStruct((_NP, _D), jnp.float32),
     jax.ShapeDtypeStruct((1, _D), jnp.float32),
     jax.ShapeDtypeStruct((1, _D), jnp.float32)],
    [_ROWS(), _VEC(), _VEC()],
)


def _call_sum3(p0, p1, x, w, b):
    out_shape, out_specs = _MM_OUT()
    return pl.pallas_call(
        _k_sum3, grid=(_GRID,),
        in_specs=[_ROWS(), _ROWS(), _ROWS(), _WMAT(), _VEC()],
        out_specs=out_specs, out_shape=out_shape,
    )(p0, p1, x, w, b)


def _call_aff(yin, a, c, w, b):
    out_shape, out_specs = _MM_OUT()
    return pl.pallas_call(
        _k_aff, grid=(_GRID,),
        in_specs=[_ROWS(), _VEC(), _VEC(), _WMAT(), _VEC()],
        out_specs=out_specs, out_shape=out_shape,
    )(yin, a, c, w, b)


def _call_dual(p0, p1, x, q0, q1, h2, wa, wb, b):
    out_shape, out_specs = _MM_OUT()
    return pl.pallas_call(
        _k_dual, grid=(_GRID,),
        in_specs=[_ROWS()] * 6 + [_WMAT(), _WMAT(), _VEC()],
        out_specs=out_specs, out_shape=out_shape,
    )(p0, p1, x, q0, q1, h2, wa, wb, b)


def _call_out(yin, a, c):
    return pl.pallas_call(
        _k_out, grid=(_GRID,),
        in_specs=[_ROWS(), _VEC(), _VEC()],
        out_specs=_ROWS(),
        out_shape=jax.ShapeDtypeStruct((_NP, _D), jnp.float32),
    )(yin, a, c)


def _affine(s, ss, g, be):
    mean = s[0] / _N
    var = ss[0] / _N - mean * mean
    scale = g / jnp.sqrt(var + _EPS)
    shift = be - mean * scale
    return scale.reshape(1, _D), shift.reshape(1, _D)


_EPAD = 323584
_EC = 128
_CHUNKS = _EPAD // (32 * _EC)
_RPS = _NP // 16

from jax.experimental.pallas import tpu_sc as plsc


def _sc_agg_body(h_hbm, src_hbm, dst_hbm, out_hbm, accum, idx_s, idx_d, rows,
                 sem):
    c = lax.axis_index("c")
    s = lax.axis_index("s")
    wid = s * 2 + c

    pltpu.sync_copy(h_hbm.at[pl.ds(s * _RPS, _RPS)],
                    accum.at[pl.ds(s * _RPS, _RPS)])
    plsc.subcore_barrier()

    base = wid * (_CHUNKS * _EC)

    def body(j, carry):
        off = base + j * _EC
        pltpu.sync_copy(src_hbm.at[pl.ds(off, _EC)], idx_s)
        pltpu.sync_copy(dst_hbm.at[pl.ds(off, _EC)], idx_d)
        pltpu.async_copy(h_hbm.at[idx_s], rows, sem).wait()
        pltpu.sync_copy(rows, accum.at[idx_d], add=True)
        return carry

    lax.fori_loop(0, _CHUNKS, body, 0)
    plsc.subcore_barrier()

    pltpu.sync_copy(accum.at[pl.ds(s * _RPS, _RPS)],
                    out_hbm.at[c, pl.ds(s * _RPS, _RPS)])


def _sc_agg(hp, srcp, dstp):
    mesh = plsc.VectorSubcoreMesh(core_axis_name="c", subcore_axis_name="s")
    f = pl.kernel(
        _sc_agg_body, mesh=mesh,
        out_type=jax.ShapeDtypeStruct((2, _NP, _D), jnp.float32),
        scratch_types=[
            pltpu.VMEM_SHARED((_NP, _D), jnp.float32),
            pltpu.VMEM((_EC,), jnp.int32),
            pltpu.VMEM((_EC,), jnp.int32),
            pltpu.VMEM((_EC, _D), jnp.float32),
            pltpu.SemaphoreType.DMA,
        ],
    )
    return f(hp, srcp, dstp)


def kernel(x, edge_index, params):
    p = params
    src = edge_index[0]
    dst = edge_index[1]

    pad = ((0, _NP - _N), (0, 0))
    xp = jnp.pad(x, pad)
    epad = _EPAD - src.shape[0]
    srcp = jnp.concatenate([src, jnp.zeros((epad,), src.dtype)])
    dstp = jnp.concatenate([dst, jnp.full((epad,), _N, dst.dtype)])

    P = _sc_agg(xp, srcp, dstp)
    p0, p1 = P[0], P[1]

    b = lambda k: p[k].reshape(1, _D)

    y1, s1, ss1 = _call_sum3(p0, p1, xp, p['W1'], b('b1'))
    sc1, sh1 = _affine(s1, ss1, p['g1'], p['be1'])
    y2, s2, ss2 = _call_aff(y1, sc1, sh1, p['W2'], b('b2'))
    sc2, sh2 = _affine(s2, ss2, p['g2'], p['be2'])
    h2 = _call_out(y2, sc2, sh2)

    Q = _sc_agg(h2, srcp, dstp)
    q0, q1 = Q[0], Q[1]

    y3, s3, ss3 = _call_dual(p0, p1, xp, q0, q1, h2,
                             p['W3'][:_D], p['W3'][_D:], b('b3'))
    sc3, sh3 = _affine(s3, ss3, p['g3'], p['be3'])
    y4, s4, ss4 = _call_aff(y3, sc3, sh3, p['W4'], b('b4'))
    sc4, sh4 = _affine(s4, ss4, p['g4'], p['be4'])
    y5, s5, ss5 = _call_aff(y4, sc4, sh4, p['W5'], b('b5'))
    sc5, sh5 = _affine(s5, ss5, p['g5'], p['be5'])
    out = _call_out(y5, sc5, sh5)
    return out[:_N]
